# Initial kernel scaffold; baseline (speedup 1.0000x reference)
import jax
import jax.numpy as jnp
from jax.experimental import pallas as pl


def kernel(verts, edges, verts_idx, edges_idx, sag0_w0, sag0_b0, sag0_w1, sag0_b1, gc0_w0, gc0_b0, gc0_w1, gc0_b1, sag1_w0, sag1_b0, sag1_w1, sag1_b1, gc1_w0, gc1_b0, gc1_w1, gc1_b1, fc1_w, fc1_b, fc2_w, fc2_b):
    raise NotImplementedError("write your pallas kernel here")


# speedup vs baseline: 20.9166x; 20.9166x over previous
import functools

import jax
import jax.numpy as jnp
from jax import lax
from jax.experimental import pallas as pl
from jax.experimental.pallas import tpu as pltpu
from jax.experimental.pallas import tpu_sc as plsc

N = 10000
E = 320000
D = 128
KEEP0 = 5000
KEEP1 = 2500
NSUB = 16
EPT = E // NSUB

F32 = jnp.float32
I32 = jnp.int32
U32 = jnp.uint32


def _mm2_body(x_ref, w_ref, b_ref, o_ref):
    o_ref[...] = (
        jnp.dot(x_ref[...], w_ref[...], preferred_element_type=F32) + b_ref[...]
    )


def _score_mm(x, w01, b01):
    return pl.pallas_call(
        _mm2_body,
        out_shape=jax.ShapeDtypeStruct((N, 2), F32),
    )(x, w01, b01)


def _sortable_u32(f):
    b = lax.bitcast_convert_type(f, U32)
    return jnp.where((b >> 31) == 1, ~b, b | U32(0x80000000))


def _topk_mask(score, k):
    key = _sortable_u32(score)
    idx = lax.broadcasted_iota(I32, (N, 1), 0)

    def vbody(i, t):
        sh = (31 - i).astype(U32)
        cand = t | (U32(1) << sh)
        cnt = jnp.sum((key >= cand).astype(I32), keepdims=True)
        return jnp.where(cnt >= k, cand, t)

    vstar = lax.fori_loop(0, 32, vbody, jnp.zeros((1, 1), U32))
    c1 = jnp.sum((key > vstar).astype(I32), keepdims=True)
    m = k - c1
    eq = key == vstar

    def ibody(i, t):
        cand = t + (I32(1) << (14 - i).astype(I32))
        cnt = jnp.sum((eq & (idx < cand)).astype(I32), keepdims=True)
        return jnp.where(cnt < m, cand, t)

    jstar = lax.fori_loop(0, 15, ibody, jnp.zeros((1, 1), I32))
    return (key > vstar) | (eq & (idx <= jstar) & (m > 0))


def _select_body(k, x_ref, sa_ref, pt_ref, w0_ref, b0_ref, w1_ref, b1_ref,
                 a_ref, bt_ref, m_ref, premask_ref=None):
    score = sa_ref[...] + pt_ref[:, 0:1] + pt_ref[:, 1:2]
    if premask_ref is not None:
        masked_score = jnp.where(premask_ref[...] > 0, score, F32(-jnp.inf))
    else:
        masked_score = score
    mask = _topk_mask(masked_score, k)
    mf = mask.astype(F32)
    x1 = jnp.maximum(x_ref[...] * jnp.tanh(score), 0.0) * mf
    a_ref[...] = jnp.dot(x1, w0_ref[...], preferred_element_type=F32) + b0_ref[...]
    bt_ref[...] = (
        jnp.dot(x1, w1_ref[...], preferred_element_type=F32) + b1_ref[...]
    ) * mf
    m_ref[...] = mf


def _select_stage(k, x, sa, pt, w0, b0, w1, b1, premask=None):
    out_shape = [
        jax.ShapeDtypeStruct((N, D), F32),
        jax.ShapeDtypeStruct((N, D), F32),
        jax.ShapeDtypeStruct((N, 1), F32),
    ]
    if premask is None:
        body = functools.partial(_select_body, k)
        return pl.pallas_call(body, out_shape=out_shape)(
            x, sa, pt, w0, b0, w1, b1)

    def body(x_ref, sa_ref, pt_ref, w0_ref, b0_ref, w1_ref, b1_ref, pm_ref,
             a_ref, bt_ref, m_ref):
        _select_body(k, x_ref, sa_ref, pt_ref, w0_ref, b0_ref, w1_ref, b1_ref,
                     a_ref, bt_ref, m_ref, premask_ref=pm_ref)

    return pl.pallas_call(body, out_shape=out_shape)(
        x, sa, pt, w0, b0, w1, b1, premask)


def _combine_body(p0_ref, p1_ref, w01_ref, b01_ref, m_ref,
                  h_ref, sa_ref, sbt_ref):
    h = jnp.maximum(jnp.concatenate([p0_ref[...], p1_ref[...]], axis=1), 0.0)
    s = jnp.dot(h, w01_ref[...], preferred_element_type=F32) + b01_ref[...]
    h_ref[...] = h
    sa_ref[...] = s[:, 0:1]
    sbt_ref[...] = s[:, 1:2] * m_ref[...]


def _combine_stage(p0, p1, w01, b01, m):
    return pl.pallas_call(
        _combine_body,
        out_shape=[
            jax.ShapeDtypeStruct((N, D), F32),
            jax.ShapeDtypeStruct((N, 1), F32),
            jax.ShapeDtypeStruct((N, 1), F32),
        ],
    )(p0, p1, w01, b01, m)


def _head_body(p0_ref, p1_ref, m_ref, fw1_ref, fb1_ref, fw2_ref, fb2_ref, o_ref):
    h2 = jnp.maximum(
        jnp.concatenate([p0_ref[...], p1_ref[...]], axis=1), 0.0) * m_ref[...]
    gap = jnp.sum(h2, axis=0, keepdims=True) * F32(1.0 / KEEP1)
    gmp = jnp.max(h2, axis=0, keepdims=True)
    cat = jnp.concatenate([gap, gmp], axis=1)
    o = jnp.maximum(
        jnp.dot(cat, fw1_ref[...], preferred_element_type=F32) + fb1_ref[...], 0.0)
    o_ref[...] = jnp.dot(o, fw2_ref[...], preferred_element_type=F32) + fb2_ref[...]


def _head_stage(p0, p1, m, fw1, fb1, fw2, fb2):
    return pl.pallas_call(
        _head_body,
        out_shape=jax.ShapeDtypeStruct((1, 55), F32),
    )(p0, p1, m, fw1, fb1, fw2, fb2)


@functools.cache
def _mesh():
    return plsc.VectorSubcoreMesh(
        core_axis_name="c", subcore_axis_name="s", num_cores=2, num_subcores=NSUB)

_SC_CHUNK = 2000
_SV_CHUNK = 400
_DRAIN_T = 10
_ROWS_PT = N // _DRAIN_T


def _scalar_agg_kernel(s_hbm, e0_hbm, e1_hbm, out_hbm,
                       di_v, si_v, vals_v, z_v, acc_sh, sem):
    c = lax.axis_index("c")
    t = lax.axis_index("s")

    @pl.when(t == 0)
    def _():
        zv = jnp.zeros((16,), F32)
        def zb(j, _):
            z_v[pl.ds(j * 16, 16)] = zv
            return 0
        lax.fori_loop(0, _SC_CHUNK // 16, zb, 0)
        def zc(j, _):
            pltpu.sync_copy(z_v, acc_sh.at[pl.ds(j * _SC_CHUNK, _SC_CHUNK)])
            return 0
        lax.fori_loop(0, N // _SC_CHUNK, zc, 0)

    plsc.subcore_barrier()

    def run_chunk(i, dst_hbm, src_hbm):
        base = t * EPT + i * _SC_CHUNK
        pltpu.sync_copy(dst_hbm.at[pl.ds(base, _SC_CHUNK)], di_v)
        pltpu.sync_copy(src_hbm.at[pl.ds(base, _SC_CHUNK)], si_v)
        pltpu.async_copy(s_hbm.at[si_v], vals_v, sem).wait()
        pltpu.sync_copy(vals_v, acc_sh.at[di_v], add=True)
        return 0

    @pl.when(c == 0)
    def _():
        lax.fori_loop(0, EPT // _SC_CHUNK, lambda i, _: run_chunk(i, e0_hbm, e1_hbm), 0)

    @pl.when(c != 0)
    def _():
        lax.fori_loop(0, EPT // _SC_CHUNK, lambda i, _: run_chunk(i, e1_hbm, e0_hbm), 0)

    plsc.subcore_barrier()

    @pl.when(t == 0)
    def _():
        pltpu.sync_copy(acc_sh, out_hbm.at[c])


def _scalar_agg(s, e0, e1):
    return pl.kernel(
        _scalar_agg_kernel,
        out_type=jax.ShapeDtypeStruct((2, N), F32),
        mesh=_mesh(),
        scratch_types=[
            pltpu.VMEM((_SC_CHUNK,), I32),
            pltpu.VMEM((_SC_CHUNK,), I32),
            pltpu.VMEM((_SC_CHUNK,), F32),
            pltpu.VMEM((_SC_CHUNK,), F32),
            pltpu.VMEM_SHARED((N,), F32),
            pltpu.SemaphoreType.DMA,
        ],
    )(s, e0, e1)


def _vec_agg_kernel(bl_hbm, br_hbm, e0_hbm, e1_hbm, al_hbm, ar_hbm,
                    ol_hbm, or_hbm, di_v, si_v, rows_v, acc_sh, sem):
    c = lax.axis_index("c")
    t = lax.axis_index("s")
    rbase = t * _ROWS_PT

    def work(b_hbm, a_hbm, o_hbm):
        @pl.when(t < _DRAIN_T)
        def _():
            pltpu.sync_copy(a_hbm.at[pl.ds(rbase, _ROWS_PT)],
                            acc_sh.at[pl.ds(rbase, _ROWS_PT)])
        plsc.subcore_barrier()

        def run_chunk(i, dst_hbm, src_hbm):
            base = t * EPT + i * _SV_CHUNK
            pltpu.sync_copy(dst_hbm.at[pl.ds(base, _SV_CHUNK)], di_v)
            pltpu.sync_copy(src_hbm.at[pl.ds(base, _SV_CHUNK)], si_v)
            pltpu.async_copy(b_hbm.at[si_v], rows_v, sem).wait()
            pltpu.sync_copy(rows_v, acc_sh.at[di_v], add=True)
            return 0

        nch = EPT // _SV_CHUNK
        lax.fori_loop(0, nch, lambda i, _: run_chunk(i, e0_hbm, e1_hbm), 0)
        lax.fori_loop(0, nch, lambda i, _: run_chunk(i, e1_hbm, e0_hbm), 0)
        plsc.subcore_barrier()

        @pl.when(t < _DRAIN_T)
        def _():
            pltpu.sync_copy(acc_sh.at[pl.ds(rbase, _ROWS_PT)],
                            o_hbm.at[pl.ds(rbase, _ROWS_PT)])

    @pl.when(c == 0)
    def _():
        work(bl_hbm, al_hbm, ol_hbm)

    @pl.when(c != 0)
    def _():
        work(br_hbm, ar_hbm, or_hbm)


def _vec_agg(bfeat, e0, e1, init):
    bl, br = bfeat[:, :D // 2], bfeat[:, D // 2:]
    al, ar = init[:, :D // 2], init[:, D // 2:]
    ol, orr = pl.kernel(
        _vec_agg_kernel,
        out_type=[
            jax.ShapeDtypeStruct((N, D // 2), F32),
            jax.ShapeDtypeStruct((N, D // 2), F32),
        ],
        mesh=_mesh(),
        scratch_types=[
            pltpu.VMEM((_SV_CHUNK,), I32),
            pltpu.VMEM((_SV_CHUNK,), I32),
            pltpu.VMEM((_SV_CHUNK, D // 2), F32),
            pltpu.VMEM_SHARED((N, D // 2), F32),
            pltpu.SemaphoreType.DMA,
        ],
        compiler_params=pltpu.CompilerParams(use_tc_tiling_on_sc=False),
    )(bl, br, e0, e1, al, ar)
    return ol, orr


def kernel(verts, edges, verts_idx, edges_idx,
           sag0_w0, sag0_b0, sag0_w1, sag0_b1,
           gc0_w0, gc0_b0, gc0_w1, gc0_b1,
           sag1_w0, sag1_b0, sag1_w1, sag1_b1,
           gc1_w0, gc1_b0, gc1_w1, gc1_b1,
           fc1_w, fc1_b, fc2_w, fc2_b):
    x = verts
    e0 = edges[:, 0]
    e1 = edges[:, 1]

    sw01 = jnp.concatenate([sag0_w0, sag0_w1], axis=1)
    sb01 = jnp.concatenate([sag0_b0, sag0_b1])[None, :]
    s0 = _score_mm(x, sw01, sb01)
    parts0 = _scalar_agg(jnp.reshape(s0[:, 1], (N,)), e0, e1)
    a0, b0t, m0 = _select_stage(
        KEEP0, x, s0[:, 0:1], parts0.T,
        gc0_w0, gc0_b0[None, :], gc0_w1, gc0_b1[None, :])

    v0a, v0b = _vec_agg(b0t, e0, e1, a0)
    sw11 = jnp.concatenate([sag1_w0, sag1_w1], axis=1)
    sb11 = jnp.concatenate([sag1_b0, sag1_b1])[None, :]
    h1, s1a, s1bt = _combine_stage(v0a, v0b, sw11, sb11, m0)
    parts1 = _scalar_agg(jnp.reshape(s1bt, (N,)), e0, e1)
    a1, b1t, m1 = _select_stage(
        KEEP1, h1, s1a, parts1.T,
        gc1_w0, gc1_b0[None, :], gc1_w1, gc1_b1[None, :], premask=m0)

    v1a, v1b = _vec_agg(b1t, e0, e1, a1)
    return _head_stage(v1a, v1b, m1, fc1_w, fc1_b[None, :], fc2_w, fc2_b[None, :])


# speedup vs baseline: 30.4131x; 1.4540x over previous
import functools

import jax
import jax.numpy as jnp
from jax import lax
from jax.experimental import pallas as pl
from jax.experimental.pallas import tpu as pltpu
from jax.experimental.pallas import tpu_sc as plsc

N = 10000
E = 320000
D = 128
KEEP0 = 5000
KEEP1 = 2500
NSUB = 16
EPT = E // NSUB

F32 = jnp.float32
I32 = jnp.int32
U32 = jnp.uint32


def _mm2_body(x_ref, w_ref, b_ref, o_ref):
    o_ref[...] = (
        jnp.dot(x_ref[...], w_ref[...], preferred_element_type=F32) + b_ref[...]
    )


def _score_mm(x, w01, b01):
    return pl.pallas_call(
        _mm2_body,
        out_shape=jax.ShapeDtypeStruct((N, 2), F32),
    )(x, w01, b01)


def _sortable_u32(f):
    b = lax.bitcast_convert_type(f, U32)
    return jnp.where((b >> 31) == 1, ~b, b | U32(0x80000000))


def _topk_mask(score, k):
    key = _sortable_u32(score)
    idx = lax.broadcasted_iota(I32, (N, 1), 0)

    def vbody(i, t):
        sh = (31 - i).astype(U32)
        cand = t | (U32(1) << sh)
        cnt = jnp.sum((key >= cand).astype(I32), keepdims=True)
        return jnp.where(cnt >= k, cand, t)

    vstar = lax.fori_loop(0, 32, vbody, jnp.zeros((1, 1), U32))
    c1 = jnp.sum((key > vstar).astype(I32), keepdims=True)
    m = k - c1
    eq = key == vstar

    def ibody(i, t):
        cand = t + (I32(1) << (14 - i).astype(I32))
        cnt = jnp.sum((eq & (idx < cand)).astype(I32), keepdims=True)
        return jnp.where(cnt < m, cand, t)

    jstar = lax.fori_loop(0, 15, ibody, jnp.zeros((1, 1), I32))
    return (key > vstar) | (eq & (idx <= jstar) & (m > 0))


def _select_body(k, x_ref, sa_ref, pt_ref, w0_ref, b0_ref, w1_ref, b1_ref,
                 a_ref, bt_ref, m_ref, premask_ref=None):
    score = sa_ref[...] + pt_ref[:, 0:1] + pt_ref[:, 1:2]
    if premask_ref is not None:
        masked_score = jnp.where(premask_ref[...] > 0, score, F32(-jnp.inf))
    else:
        masked_score = score
    mask = _topk_mask(masked_score, k)
    mf = mask.astype(F32)
    x1 = jnp.maximum(x_ref[...] * jnp.tanh(score), 0.0) * mf
    a_ref[...] = jnp.dot(x1, w0_ref[...], preferred_element_type=F32) + b0_ref[...]
    bt_ref[...] = (
        jnp.dot(x1, w1_ref[...], preferred_element_type=F32) + b1_ref[...]
    ) * mf
    m_ref[...] = mf


def _select_stage(k, x, sa, pt, w0, b0, w1, b1, premask=None):
    out_shape = [
        jax.ShapeDtypeStruct((N, D), F32),
        jax.ShapeDtypeStruct((N, D), F32),
        jax.ShapeDtypeStruct((N, 1), F32),
    ]
    if premask is None:
        body = functools.partial(_select_body, k)
        return pl.pallas_call(body, out_shape=out_shape)(
            x, sa, pt, w0, b0, w1, b1)

    def body(x_ref, sa_ref, pt_ref, w0_ref, b0_ref, w1_ref, b1_ref, pm_ref,
             a_ref, bt_ref, m_ref):
        _select_body(k, x_ref, sa_ref, pt_ref, w0_ref, b0_ref, w1_ref, b1_ref,
                     a_ref, bt_ref, m_ref, premask_ref=pm_ref)

    return pl.pallas_call(body, out_shape=out_shape)(
        x, sa, pt, w0, b0, w1, b1, premask)


def _combine_body(p0_ref, p1_ref, w01_ref, b01_ref, m_ref,
                  h_ref, sa_ref, sbt_ref):
    h = jnp.maximum(jnp.concatenate([p0_ref[...], p1_ref[...]], axis=1), 0.0)
    s = jnp.dot(h, w01_ref[...], preferred_element_type=F32) + b01_ref[...]
    h_ref[...] = h
    sa_ref[...] = s[:, 0:1]
    sbt_ref[...] = s[:, 1:2] * m_ref[...]


def _combine_stage(p0, p1, w01, b01, m):
    return pl.pallas_call(
        _combine_body,
        out_shape=[
            jax.ShapeDtypeStruct((N, D), F32),
            jax.ShapeDtypeStruct((N, 1), F32),
            jax.ShapeDtypeStruct((N, 1), F32),
        ],
    )(p0, p1, w01, b01, m)


def _head_body(p0_ref, p1_ref, m_ref, fw1_ref, fb1_ref, fw2_ref, fb2_ref, o_ref):
    h2 = jnp.maximum(
        jnp.concatenate([p0_ref[...], p1_ref[...]], axis=1), 0.0) * m_ref[...]
    gap = jnp.sum(h2, axis=0, keepdims=True) * F32(1.0 / KEEP1)
    gmp = jnp.max(h2, axis=0, keepdims=True)
    cat = jnp.concatenate([gap, gmp], axis=1)
    o = jnp.maximum(
        jnp.dot(cat, fw1_ref[...], preferred_element_type=F32) + fb1_ref[...], 0.0)
    o_ref[...] = jnp.dot(o, fw2_ref[...], preferred_element_type=F32) + fb2_ref[...]


def _head_stage(p0, p1, m, fw1, fb1, fw2, fb2):
    return pl.pallas_call(
        _head_body,
        out_shape=jax.ShapeDtypeStruct((1, 55), F32),
    )(p0, p1, m, fw1, fb1, fw2, fb2)


@functools.cache
def _mesh():
    return plsc.VectorSubcoreMesh(
        core_axis_name="c", subcore_axis_name="s", num_cores=2, num_subcores=NSUB)

_SC_CHUNK = 2000
_SV_CHUNK = 400
_SUP = 10000
_DRAIN_T = 10
_ROWS_PT = N // _DRAIN_T


def _scalar_agg_kernel(s_hbm, e0_hbm, e1_hbm, out_hbm,
                       di_v, si_v, vals_v, z_v, acc_sh, sem):
    c = lax.axis_index("c")
    t = lax.axis_index("s")

    @pl.when(t == 0)
    def _():
        zv = jnp.zeros((16,), F32)
        def zb(j, _):
            z_v[pl.ds(j * 16, 16)] = zv
            return 0
        lax.fori_loop(0, _SC_CHUNK // 16, zb, 0)
        def zc(j, _):
            pltpu.sync_copy(z_v, acc_sh.at[pl.ds(j * _SC_CHUNK, _SC_CHUNK)])
            return 0
        lax.fori_loop(0, N // _SC_CHUNK, zc, 0)

    plsc.subcore_barrier()

    def run_chunk(i, dst_hbm, src_hbm):
        base = t * EPT + i * _SC_CHUNK
        pltpu.sync_copy(dst_hbm.at[pl.ds(base, _SC_CHUNK)], di_v)
        pltpu.sync_copy(src_hbm.at[pl.ds(base, _SC_CHUNK)], si_v)
        pltpu.async_copy(s_hbm.at[si_v], vals_v, sem).wait()
        pltpu.sync_copy(vals_v, acc_sh.at[di_v], add=True)
        return 0

    @pl.when(c == 0)
    def _():
        lax.fori_loop(0, EPT // _SC_CHUNK, lambda i, _: run_chunk(i, e0_hbm, e1_hbm), 0)

    @pl.when(c != 0)
    def _():
        lax.fori_loop(0, EPT // _SC_CHUNK, lambda i, _: run_chunk(i, e1_hbm, e0_hbm), 0)

    plsc.subcore_barrier()

    @pl.when(t == 0)
    def _():
        pltpu.sync_copy(acc_sh, out_hbm.at[c])


def _scalar_agg(s, e0, e1):
    return pl.kernel(
        _scalar_agg_kernel,
        out_type=jax.ShapeDtypeStruct((2, N), F32),
        mesh=_mesh(),
        scratch_types=[
            pltpu.VMEM((_SC_CHUNK,), I32),
            pltpu.VMEM((_SC_CHUNK,), I32),
            pltpu.VMEM((_SC_CHUNK,), F32),
            pltpu.VMEM((_SC_CHUNK,), F32),
            pltpu.VMEM_SHARED((N,), F32),
            pltpu.SemaphoreType.DMA,
        ],
    )(s, e0, e1)


def _vec_agg_kernel(bl_hbm, br_hbm, e0_hbm, e1_hbm, al_hbm, ar_hbm,
                    ol_hbm, or_hbm, i0_v, i1_v, rows_v, acc_sh, gsem, ssem):
    c = lax.axis_index("c")
    t = lax.axis_index("s")
    rbase = t * _ROWS_PT

    def work(b_hbm, a_hbm, o_hbm):
        @pl.when(t < _DRAIN_T)
        def _():
            pltpu.sync_copy(a_hbm.at[pl.ds(rbase, _ROWS_PT)],
                            acc_sh.at[pl.ds(rbase, _ROWS_PT)])
        plsc.subcore_barrier()

        nch = _SUP // _SV_CHUNK
        total = 2 * nch

        def gather(i, slot):
            j = jnp.where(i < nch, i, i - nch) * _SV_CHUNK
            @pl.when(i < nch)
            def _():
                pltpu.async_copy(
                    b_hbm.at[i1_v.at[pl.ds(j, _SV_CHUNK)]],
                    rows_v.at[slot], gsem)
            @pl.when(i >= nch)
            def _():
                pltpu.async_copy(
                    b_hbm.at[i0_v.at[pl.ds(j, _SV_CHUNK)]],
                    rows_v.at[slot], gsem)

        def scatter(i, slot):
            j = jnp.where(i < nch, i, i - nch) * _SV_CHUNK
            @pl.when(i < nch)
            def _():
                pltpu.async_copy(rows_v.at[slot],
                                 acc_sh.at[i0_v.at[pl.ds(j, _SV_CHUNK)]],
                                 ssem, add=True)
            @pl.when(i >= nch)
            def _():
                pltpu.async_copy(rows_v.at[slot],
                                 acc_sh.at[i1_v.at[pl.ds(j, _SV_CHUNK)]],
                                 ssem, add=True)

        def wait_gather(slot):
            pltpu.make_async_copy(b_hbm.at[i0_v.at[pl.ds(0, _SV_CHUNK)]],
                                  rows_v.at[slot], gsem).wait()

        def wait_scatter(slot):
            pltpu.make_async_copy(rows_v.at[slot],
                                  acc_sh.at[i0_v.at[pl.ds(0, _SV_CHUNK)]],
                                  ssem).wait()

        for s in range(EPT // _SUP):
            base = t * EPT + s * _SUP
            pltpu.sync_copy(e0_hbm.at[pl.ds(base, _SUP)], i0_v)
            pltpu.sync_copy(e1_hbm.at[pl.ds(base, _SUP)], i1_v)
            gather(0, 0)
            gather(1, 1)

            def step(i, _):
                slot = lax.rem(i, 2)
                wait_gather(slot)
                scatter(i, slot)
                @pl.when(i + 2 < total)
                def _():
                    wait_scatter(slot)
                    gather(i + 2, slot)
                return 0

            lax.fori_loop(0, total, step, 0)
            wait_scatter(0)
            wait_scatter(1)

        plsc.subcore_barrier()

        @pl.when(t < _DRAIN_T)
        def _():
            pltpu.sync_copy(acc_sh.at[pl.ds(rbase, _ROWS_PT)],
                            o_hbm.at[pl.ds(rbase, _ROWS_PT)])

    @pl.when(c == 0)
    def _():
        work(bl_hbm, al_hbm, ol_hbm)

    @pl.when(c != 0)
    def _():
        work(br_hbm, ar_hbm, or_hbm)


def _vec_agg(bfeat, e0, e1, init):
    bl, br = bfeat[:, :D // 2], bfeat[:, D // 2:]
    al, ar = init[:, :D // 2], init[:, D // 2:]
    ol, orr = pl.kernel(
        _vec_agg_kernel,
        out_type=[
            jax.ShapeDtypeStruct((N, D // 2), F32),
            jax.ShapeDtypeStruct((N, D // 2), F32),
        ],
        mesh=_mesh(),
        scratch_types=[
            pltpu.VMEM((_SUP,), I32),
            pltpu.VMEM((_SUP,), I32),
            pltpu.VMEM((2, _SV_CHUNK, D // 2), F32),
            pltpu.VMEM_SHARED((N, D // 2), F32),
            pltpu.SemaphoreType.DMA,
            pltpu.SemaphoreType.DMA,
        ],
        compiler_params=pltpu.CompilerParams(use_tc_tiling_on_sc=False),
    )(bl, br, e0, e1, al, ar)
    return ol, orr


def kernel(verts, edges, verts_idx, edges_idx,
           sag0_w0, sag0_b0, sag0_w1, sag0_b1,
           gc0_w0, gc0_b0, gc0_w1, gc0_b1,
           sag1_w0, sag1_b0, sag1_w1, sag1_b1,
           gc1_w0, gc1_b0, gc1_w1, gc1_b1,
           fc1_w, fc1_b, fc2_w, fc2_b):
    x = verts
    e0 = edges[:, 0]
    e1 = edges[:, 1]

    sw01 = jnp.concatenate([sag0_w0, sag0_w1], axis=1)
    sb01 = jnp.concatenate([sag0_b0, sag0_b1])[None, :]
    s0 = _score_mm(x, sw01, sb01)
    parts0 = _scalar_agg(jnp.reshape(s0[:, 1], (N,)), e0, e1)
    a0, b0t, m0 = _select_stage(
        KEEP0, x, s0[:, 0:1], parts0.T,
        gc0_w0, gc0_b0[None, :], gc0_w1, gc0_b1[None, :])

    v0a, v0b = _vec_agg(b0t, e0, e1, a0)
    sw11 = jnp.concatenate([sag1_w0, sag1_w1], axis=1)
    sb11 = jnp.concatenate([sag1_b0, sag1_b1])[None, :]
    h1, s1a, s1bt = _combine_stage(v0a, v0b, sw11, sb11, m0)
    parts1 = _scalar_agg(jnp.reshape(s1bt, (N,)), e0, e1)
    a1, b1t, m1 = _select_stage(
        KEEP1, h1, s1a, parts1.T,
        gc1_w0, gc1_b0[None, :], gc1_w1, gc1_b1[None, :], premask=m0)

    v1a, v1b = _vec_agg(b1t, e0, e1, a1)
    return _head_stage(v1a, v1b, m1, fc1_w, fc1_b[None, :], fc2_w, fc2_b[None, :])


# speedup vs baseline: 30.4579x; 1.0015x over previous
import functools

import jax
import jax.numpy as jnp
from jax import lax
from jax.experimental import pallas as pl
from jax.experimental.pallas import tpu as pltpu
from jax.experimental.pallas import tpu_sc as plsc

N = 10000
E = 320000
D = 128
KEEP0 = 5000
KEEP1 = 2500
NSUB = 16
EPT = E // NSUB

F32 = jnp.float32
I32 = jnp.int32
U32 = jnp.uint32


def _mm2_body(x_ref, w_ref, b_ref, o_ref):
    o_ref[...] = (
        jnp.dot(x_ref[...], w_ref[...], preferred_element_type=F32) + b_ref[...]
    )


def _score_mm(x, w01, b01):
    return pl.pallas_call(
        _mm2_body,
        out_shape=jax.ShapeDtypeStruct((N, 2), F32),
    )(x, w01, b01)


def _sortable_u32(f):
    b = lax.bitcast_convert_type(f, U32)
    return jnp.where((b >> 31) == 1, ~b, b | U32(0x80000000))


def _topk_mask(score, k):
    key = _sortable_u32(score)
    idx = lax.broadcasted_iota(I32, (N, 1), 0)

    def vbody(i, t):
        sh = (31 - i).astype(U32)
        cand = t | (U32(1) << sh)
        cnt = jnp.sum((key >= cand).astype(I32), keepdims=True)
        return jnp.where(cnt >= k, cand, t)

    vstar = lax.fori_loop(0, 32, vbody, jnp.zeros((1, 1), U32))
    c1 = jnp.sum((key > vstar).astype(I32), keepdims=True)
    m = k - c1
    eq = key == vstar

    def ibody(i, t):
        cand = t + (I32(1) << (14 - i).astype(I32))
        cnt = jnp.sum((eq & (idx < cand)).astype(I32), keepdims=True)
        return jnp.where(cnt < m, cand, t)

    jstar = lax.fori_loop(0, 15, ibody, jnp.zeros((1, 1), I32))
    return (key > vstar) | (eq & (idx <= jstar) & (m > 0))


def _select_body(k, x_ref, sa_ref, pt_ref, w0_ref, b0_ref, w1_ref, b1_ref,
                 a_ref, bt_ref, m_ref, premask_ref=None):
    score = sa_ref[...] + pt_ref[:, 0:1] + pt_ref[:, 1:2]
    if premask_ref is not None:
        masked_score = jnp.where(premask_ref[...] > 0, score, F32(-jnp.inf))
    else:
        masked_score = score
    mask = _topk_mask(masked_score, k)
    mf = mask.astype(F32)
    x1 = jnp.maximum(x_ref[...] * jnp.tanh(score), 0.0) * mf
    a_ref[...] = jnp.dot(x1, w0_ref[...], preferred_element_type=F32) + b0_ref[...]
    bt_ref[...] = (
        jnp.dot(x1, w1_ref[...], preferred_element_type=F32) + b1_ref[...]
    ) * mf
    m_ref[...] = mf


def _select_stage(k, x, sa, pt, w0, b0, w1, b1, premask=None):
    out_shape = [
        jax.ShapeDtypeStruct((N, D), F32),
        jax.ShapeDtypeStruct((N, D), F32),
        jax.ShapeDtypeStruct((N, 1), F32),
    ]
    if premask is None:
        body = functools.partial(_select_body, k)
        return pl.pallas_call(body, out_shape=out_shape)(
            x, sa, pt, w0, b0, w1, b1)

    def body(x_ref, sa_ref, pt_ref, w0_ref, b0_ref, w1_ref, b1_ref, pm_ref,
             a_ref, bt_ref, m_ref):
        _select_body(k, x_ref, sa_ref, pt_ref, w0_ref, b0_ref, w1_ref, b1_ref,
                     a_ref, bt_ref, m_ref, premask_ref=pm_ref)

    return pl.pallas_call(body, out_shape=out_shape)(
        x, sa, pt, w0, b0, w1, b1, premask)


def _combine_body(p0_ref, p1_ref, w01_ref, b01_ref, m_ref,
                  h_ref, sa_ref, sbt_ref):
    h = jnp.maximum(jnp.concatenate([p0_ref[...], p1_ref[...]], axis=1), 0.0)
    s = jnp.dot(h, w01_ref[...], preferred_element_type=F32) + b01_ref[...]
    h_ref[...] = h
    sa_ref[...] = s[:, 0:1]
    sbt_ref[...] = s[:, 1:2] * m_ref[...]


def _combine_stage(p0, p1, w01, b01, m):
    return pl.pallas_call(
        _combine_body,
        out_shape=[
            jax.ShapeDtypeStruct((N, D), F32),
            jax.ShapeDtypeStruct((N, 1), F32),
            jax.ShapeDtypeStruct((N, 1), F32),
        ],
    )(p0, p1, w01, b01, m)


def _head_body(p0_ref, p1_ref, m_ref, fw1_ref, fb1_ref, fw2_ref, fb2_ref, o_ref):
    h2 = jnp.maximum(
        jnp.concatenate([p0_ref[...], p1_ref[...]], axis=1), 0.0) * m_ref[...]
    gap = jnp.sum(h2, axis=0, keepdims=True) * F32(1.0 / KEEP1)
    gmp = jnp.max(h2, axis=0, keepdims=True)
    cat = jnp.concatenate([gap, gmp], axis=1)
    o = jnp.maximum(
        jnp.dot(cat, fw1_ref[...], preferred_element_type=F32) + fb1_ref[...], 0.0)
    o_ref[...] = jnp.dot(o, fw2_ref[...], preferred_element_type=F32) + fb2_ref[...]


def _head_stage(p0, p1, m, fw1, fb1, fw2, fb2):
    return pl.pallas_call(
        _head_body,
        out_shape=jax.ShapeDtypeStruct((1, 55), F32),
    )(p0, p1, m, fw1, fb1, fw2, fb2)


@functools.cache
def _mesh():
    return plsc.VectorSubcoreMesh(
        core_axis_name="c", subcore_axis_name="s", num_cores=2, num_subcores=NSUB)

_SC_CHUNK = 2000
_SV_CHUNK = 400
_SUP = 10000
_DRAIN_T = 10
_ROWS_PT = N // _DRAIN_T


def _scalar_agg_kernel(s_hbm, e0_hbm, e1_hbm, out_hbm,
                       di_v, si_v, vals_v, z_v, acc_sh, gsem, ssem):
    c = lax.axis_index("c")
    t = lax.axis_index("s")

    @pl.when(t == 0)
    def _():
        zv = jnp.zeros((16,), F32)
        def zb(j, _):
            z_v[pl.ds(j * 16, 16)] = zv
            return 0
        lax.fori_loop(0, _SC_CHUNK // 16, zb, 0)
        def zc(j, _):
            pltpu.sync_copy(z_v, acc_sh.at[pl.ds(j * _SC_CHUNK, _SC_CHUNK)])
            return 0
        lax.fori_loop(0, N // _SC_CHUNK, zc, 0)

    def work(dst_hbm, src_hbm):
        pltpu.sync_copy(dst_hbm.at[pl.ds(t * EPT, EPT)], di_v)
        pltpu.sync_copy(src_hbm.at[pl.ds(t * EPT, EPT)], si_v)
        plsc.subcore_barrier()
        nch = EPT // _SC_CHUNK

        def gather(i, slot):
            pltpu.async_copy(s_hbm.at[si_v.at[pl.ds(i * _SC_CHUNK, _SC_CHUNK)]],
                             vals_v.at[slot], gsem)

        def scatter(i, slot):
            pltpu.async_copy(vals_v.at[slot],
                             acc_sh.at[di_v.at[pl.ds(i * _SC_CHUNK, _SC_CHUNK)]],
                             ssem, add=True)

        def wait_gather(slot):
            pltpu.make_async_copy(
                s_hbm.at[si_v.at[pl.ds(0, _SC_CHUNK)]],
                vals_v.at[slot], gsem).wait()

        def wait_scatter(slot):
            pltpu.make_async_copy(
                vals_v.at[slot],
                acc_sh.at[di_v.at[pl.ds(0, _SC_CHUNK)]], ssem).wait()

        gather(0, 0)
        gather(1, 1)

        def step(i, _):
            slot = lax.rem(i, 2)
            wait_gather(slot)
            scatter(i, slot)
            @pl.when(i + 2 < nch)
            def _():
                wait_scatter(slot)
                gather(i + 2, slot)
            return 0

        lax.fori_loop(0, nch, step, 0)
        wait_scatter(0)
        wait_scatter(1)

    @pl.when(c == 0)
    def _():
        work(e0_hbm, e1_hbm)

    @pl.when(c != 0)
    def _():
        work(e1_hbm, e0_hbm)

    plsc.subcore_barrier()

    @pl.when(t == 0)
    def _():
        pltpu.sync_copy(acc_sh, out_hbm.at[c])


def _scalar_agg(s, e0, e1):
    return pl.kernel(
        _scalar_agg_kernel,
        out_type=jax.ShapeDtypeStruct((2, N), F32),
        mesh=_mesh(),
        scratch_types=[
            pltpu.VMEM((EPT,), I32),
            pltpu.VMEM((EPT,), I32),
            pltpu.VMEM((2, _SC_CHUNK), F32),
            pltpu.VMEM((_SC_CHUNK,), F32),
            pltpu.VMEM_SHARED((N,), F32),
            pltpu.SemaphoreType.DMA,
            pltpu.SemaphoreType.DMA,
        ],
        compiler_params=pltpu.CompilerParams(use_tc_tiling_on_sc=False),
    )(s, e0, e1)


def _vec_agg_kernel(bl_hbm, br_hbm, e0_hbm, e1_hbm, al_hbm, ar_hbm,
                    ol_hbm, or_hbm, i0_v, i1_v, rows_v, acc_sh, gsem, ssem):
    c = lax.axis_index("c")
    t = lax.axis_index("s")
    rbase = t * _ROWS_PT

    def work(b_hbm, a_hbm, o_hbm):
        @pl.when(t < _DRAIN_T)
        def _():
            pltpu.sync_copy(a_hbm.at[pl.ds(rbase, _ROWS_PT)],
                            acc_sh.at[pl.ds(rbase, _ROWS_PT)])
        plsc.subcore_barrier()

        nch = _SUP // _SV_CHUNK
        total = 2 * nch

        def gather(i, slot):
            j = jnp.where(i < nch, i, i - nch) * _SV_CHUNK
            @pl.when(i < nch)
            def _():
                pltpu.async_copy(
                    b_hbm.at[i1_v.at[pl.ds(j, _SV_CHUNK)]],
                    rows_v.at[slot], gsem)
            @pl.when(i >= nch)
            def _():
                pltpu.async_copy(
                    b_hbm.at[i0_v.at[pl.ds(j, _SV_CHUNK)]],
                    rows_v.at[slot], gsem)

        def scatter(i, slot):
            j = jnp.where(i < nch, i, i - nch) * _SV_CHUNK
            @pl.when(i < nch)
            def _():
                pltpu.async_copy(rows_v.at[slot],
                                 acc_sh.at[i0_v.at[pl.ds(j, _SV_CHUNK)]],
                                 ssem, add=True)
            @pl.when(i >= nch)
            def _():
                pltpu.async_copy(rows_v.at[slot],
                                 acc_sh.at[i1_v.at[pl.ds(j, _SV_CHUNK)]],
                                 ssem, add=True)

        def wait_gather(slot):
            pltpu.make_async_copy(b_hbm.at[i0_v.at[pl.ds(0, _SV_CHUNK)]],
                                  rows_v.at[slot], gsem).wait()

        def wait_scatter(slot):
            pltpu.make_async_copy(rows_v.at[slot],
                                  acc_sh.at[i0_v.at[pl.ds(0, _SV_CHUNK)]],
                                  ssem).wait()

        for s in range(EPT // _SUP):
            base = t * EPT + s * _SUP
            pltpu.sync_copy(e0_hbm.at[pl.ds(base, _SUP)], i0_v)
            pltpu.sync_copy(e1_hbm.at[pl.ds(base, _SUP)], i1_v)
            gather(0, 0)
            gather(1, 1)

            def step(i, _):
                slot = lax.rem(i, 2)
                wait_gather(slot)
                scatter(i, slot)
                @pl.when(i + 2 < total)
                def _():
                    wait_scatter(slot)
                    gather(i + 2, slot)
                return 0

            lax.fori_loop(0, total, step, 0)
            wait_scatter(0)
            wait_scatter(1)

        plsc.subcore_barrier()

        @pl.when(t < _DRAIN_T)
        def _():
            pltpu.sync_copy(acc_sh.at[pl.ds(rbase, _ROWS_PT)],
                            o_hbm.at[pl.ds(rbase, _ROWS_PT)])

    @pl.when(c == 0)
    def _():
        work(bl_hbm, al_hbm, ol_hbm)

    @pl.when(c != 0)
    def _():
        work(br_hbm, ar_hbm, or_hbm)


def _vec_agg(bfeat, e0, e1, init):
    bl, br = bfeat[:, :D // 2], bfeat[:, D // 2:]
    al, ar = init[:, :D // 2], init[:, D // 2:]
    ol, orr = pl.kernel(
        _vec_agg_kernel,
        out_type=[
            jax.ShapeDtypeStruct((N, D // 2), F32),
            jax.ShapeDtypeStruct((N, D // 2), F32),
        ],
        mesh=_mesh(),
        scratch_types=[
            pltpu.VMEM((_SUP,), I32),
            pltpu.VMEM((_SUP,), I32),
            pltpu.VMEM((2, _SV_CHUNK, D // 2), F32),
            pltpu.VMEM_SHARED((N, D // 2), F32),
            pltpu.SemaphoreType.DMA,
            pltpu.SemaphoreType.DMA,
        ],
        compiler_params=pltpu.CompilerParams(use_tc_tiling_on_sc=False),
    )(bl, br, e0, e1, al, ar)
    return ol, orr


def kernel(verts, edges, verts_idx, edges_idx,
           sag0_w0, sag0_b0, sag0_w1, sag0_b1,
           gc0_w0, gc0_b0, gc0_w1, gc0_b1,
           sag1_w0, sag1_b0, sag1_w1, sag1_b1,
           gc1_w0, gc1_b0, gc1_w1, gc1_b1,
           fc1_w, fc1_b, fc2_w, fc2_b):
    x = verts
    e0 = edges[:, 0]
    e1 = edges[:, 1]

    sw01 = jnp.concatenate([sag0_w0, sag0_w1], axis=1)
    sb01 = jnp.concatenate([sag0_b0, sag0_b1])[None, :]
    s0 = _score_mm(x, sw01, sb01)
    parts0 = _scalar_agg(jnp.reshape(s0[:, 1], (N,)), e0, e1)
    a0, b0t, m0 = _select_stage(
        KEEP0, x, s0[:, 0:1], parts0.T,
        gc0_w0, gc0_b0[None, :], gc0_w1, gc0_b1[None, :])

    v0a, v0b = _vec_agg(b0t, e0, e1, a0)
    sw11 = jnp.concatenate([sag1_w0, sag1_w1], axis=1)
    sb11 = jnp.concatenate([sag1_b0, sag1_b1])[None, :]
    h1, s1a, s1bt = _combine_stage(v0a, v0b, sw11, sb11, m0)
    parts1 = _scalar_agg(jnp.reshape(s1bt, (N,)), e0, e1)
    a1, b1t, m1 = _select_stage(
        KEEP1, h1, s1a, parts1.T,
        gc1_w0, gc1_b0[None, :], gc1_w1, gc1_b1[None, :], premask=m0)

    v1a, v1b = _vec_agg(b1t, e0, e1, a1)
    return _head_stage(v1a, v1b, m1, fc1_w, fc1_b[None, :], fc2_w, fc2_b[None, :])


# speedup vs baseline: 35.1079x; 1.1527x over previous
import functools

import jax
import jax.numpy as jnp
from jax import lax
from jax.experimental import pallas as pl
from jax.experimental.pallas import tpu as pltpu
from jax.experimental.pallas import tpu_sc as plsc

N = 10000
E = 320000
D = 128
KEEP0 = 5000
KEEP1 = 2500
NSUB = 16
EPT = E // NSUB

F32 = jnp.float32
I32 = jnp.int32
U32 = jnp.uint32


def _mm2_body(x_ref, w_ref, b_ref, o_ref):
    o_ref[...] = (
        jnp.dot(x_ref[...], w_ref[...], preferred_element_type=F32) + b_ref[...]
    )


def _score_mm(x, w01, b01):
    return pl.pallas_call(
        _mm2_body,
        out_shape=jax.ShapeDtypeStruct((N, 2), F32),
    )(x, w01, b01)


def _sortable_u32(f):
    b = lax.bitcast_convert_type(f, U32)
    return jnp.where((b >> 31) == 1, ~b, b | U32(0x80000000))


def _topk_mask(score, k):
    key = _sortable_u32(score)
    idx = lax.broadcasted_iota(I32, (N, 1), 0)

    def vbody(i, t):
        sh = (31 - i).astype(U32)
        cand = t | (U32(1) << sh)
        cnt = jnp.sum((key >= cand).astype(I32), keepdims=True)
        return jnp.where(cnt >= k, cand, t)

    vstar = lax.fori_loop(0, 32, vbody, jnp.zeros((1, 1), U32))
    c1 = jnp.sum((key > vstar).astype(I32), keepdims=True)
    m = k - c1
    eq = key == vstar

    def ibody(i, t):
        cand = t + (I32(1) << (14 - i).astype(I32))
        cnt = jnp.sum((eq & (idx < cand)).astype(I32), keepdims=True)
        return jnp.where(cnt < m, cand, t)

    jstar = lax.fori_loop(0, 15, ibody, jnp.zeros((1, 1), I32))
    return (key > vstar) | (eq & (idx <= jstar) & (m > 0))


def _select_body(k, x_ref, sa_ref, pt_ref, w0_ref, b0_ref, w1_ref, b1_ref,
                 a_ref, bt_ref, m_ref, premask_ref=None):
    score = sa_ref[...] + pt_ref[:, 0:1] + pt_ref[:, 1:2]
    if premask_ref is not None:
        masked_score = jnp.where(premask_ref[...] > 0, score, F32(-jnp.inf))
    else:
        masked_score = score
    mask = _topk_mask(masked_score, k)
    mf = mask.astype(F32)
    x1 = jnp.maximum(x_ref[...] * jnp.tanh(score), 0.0) * mf
    a_ref[...] = jnp.dot(x1, w0_ref[...], preferred_element_type=F32) + b0_ref[...]
    bt_ref[...] = (
        jnp.dot(x1, w1_ref[...], preferred_element_type=F32) + b1_ref[...]
    ) * mf
    m_ref[...] = mf


def _select_stage(k, x, sa, pt, w0, b0, w1, b1, premask=None):
    out_shape = [
        jax.ShapeDtypeStruct((N, D), F32),
        jax.ShapeDtypeStruct((N, D), F32),
        jax.ShapeDtypeStruct((N, 1), F32),
    ]
    if premask is None:
        body = functools.partial(_select_body, k)
        return pl.pallas_call(body, out_shape=out_shape)(
            x, sa, pt, w0, b0, w1, b1)

    def body(x_ref, sa_ref, pt_ref, w0_ref, b0_ref, w1_ref, b1_ref, pm_ref,
             a_ref, bt_ref, m_ref):
        _select_body(k, x_ref, sa_ref, pt_ref, w0_ref, b0_ref, w1_ref, b1_ref,
                     a_ref, bt_ref, m_ref, premask_ref=pm_ref)

    return pl.pallas_call(body, out_shape=out_shape)(
        x, sa, pt, w0, b0, w1, b1, premask)


def _combine_body(p0_ref, p1_ref, w01_ref, b01_ref, m_ref,
                  h_ref, sa_ref, sbt_ref):
    h = jnp.maximum(jnp.concatenate([p0_ref[...], p1_ref[...]], axis=1), 0.0)
    s = jnp.dot(h, w01_ref[...], preferred_element_type=F32) + b01_ref[...]
    h_ref[...] = h
    sa_ref[...] = s[:, 0:1]
    sbt_ref[...] = s[:, 1:2] * m_ref[...]


def _combine_stage(p0, p1, w01, b01, m):
    return pl.pallas_call(
        _combine_body,
        out_shape=[
            jax.ShapeDtypeStruct((N, D), F32),
            jax.ShapeDtypeStruct((N, 1), F32),
            jax.ShapeDtypeStruct((N, 1), F32),
        ],
    )(p0, p1, w01, b01, m)


def _head_body(p0_ref, p1_ref, m_ref, fw1_ref, fb1_ref, fw2_ref, fb2_ref, o_ref):
    h2 = jnp.maximum(
        jnp.concatenate([p0_ref[...], p1_ref[...]], axis=1), 0.0) * m_ref[...]
    gap = jnp.sum(h2, axis=0, keepdims=True) * F32(1.0 / KEEP1)
    gmp = jnp.max(h2, axis=0, keepdims=True)
    cat = jnp.concatenate([gap, gmp], axis=1)
    o = jnp.maximum(
        jnp.dot(cat, fw1_ref[...], preferred_element_type=F32) + fb1_ref[...], 0.0)
    o_ref[...] = jnp.dot(o, fw2_ref[...], preferred_element_type=F32) + fb2_ref[...]


def _head_stage(p0, p1, m, fw1, fb1, fw2, fb2):
    return pl.pallas_call(
        _head_body,
        out_shape=jax.ShapeDtypeStruct((1, 55), F32),
    )(p0, p1, m, fw1, fb1, fw2, fb2)


@functools.cache
def _mesh():
    return plsc.VectorSubcoreMesh(
        core_axis_name="c", subcore_axis_name="s", num_cores=2, num_subcores=NSUB)

_SC_CHUNK = 2000
_SV_CHUNK = 400
_SUP = 10000
_DRAIN_T = 10
_ROWS_PT = N // _DRAIN_T


def _scalar_agg_kernel(s_hbm, e0_hbm, e1_hbm, out_hbm,
                       di_v, si_v, vals_v, z_v, s_v, acc_sh):
    c = lax.axis_index("c")
    t = lax.axis_index("s")

    @pl.when(t == 0)
    def _():
        zv = jnp.zeros((16,), F32)
        def zb(j, _):
            z_v[pl.ds(j * 16, 16)] = zv
            return 0
        lax.fori_loop(0, _SC_CHUNK // 16, zb, 0)
        def zc(j, _):
            pltpu.sync_copy(z_v, acc_sh.at[pl.ds(j * _SC_CHUNK, _SC_CHUNK)])
            return 0
        lax.fori_loop(0, N // _SC_CHUNK, zc, 0)

    def work(dst_hbm, src_hbm):
        pltpu.sync_copy(s_hbm, s_v)
        pltpu.sync_copy(dst_hbm.at[pl.ds(t * EPT, EPT)], di_v)
        pltpu.sync_copy(src_hbm.at[pl.ds(t * EPT, EPT)], si_v)
        plsc.subcore_barrier()
        nch = EPT // _SC_CHUNK

        def chunk(i, _):
            def gb(j, _):
                ii = si_v[pl.ds(i * _SC_CHUNK + j * 16, 16)]
                vals_v[pl.ds(j * 16, 16)] = plsc.load_gather(s_v, [ii])
                return 0
            lax.fori_loop(0, _SC_CHUNK // 16, gb, 0)
            pltpu.sync_copy(vals_v,
                            acc_sh.at[di_v.at[pl.ds(i * _SC_CHUNK, _SC_CHUNK)]],
                            add=True)
            return 0

        lax.fori_loop(0, nch, chunk, 0)

    @pl.when(c == 0)
    def _():
        work(e0_hbm, e1_hbm)

    @pl.when(c != 0)
    def _():
        work(e1_hbm, e0_hbm)

    plsc.subcore_barrier()

    @pl.when(t == 0)
    def _():
        pltpu.sync_copy(acc_sh, out_hbm.at[c])


def _scalar_agg(s, e0, e1):
    return pl.kernel(
        _scalar_agg_kernel,
        out_type=jax.ShapeDtypeStruct((2, N), F32),
        mesh=_mesh(),
        scratch_types=[
            pltpu.VMEM((EPT,), I32),
            pltpu.VMEM((EPT,), I32),
            pltpu.VMEM((_SC_CHUNK,), F32),
            pltpu.VMEM((_SC_CHUNK,), F32),
            pltpu.VMEM((N,), F32),
            pltpu.VMEM_SHARED((N,), F32),
        ],
        compiler_params=pltpu.CompilerParams(
            use_tc_tiling_on_sc=False, needs_layout_passes=False),
    )(s, e0, e1)


def _vec_agg_kernel(bl_hbm, br_hbm, e0_hbm, e1_hbm, al_hbm, ar_hbm,
                    ol_hbm, or_hbm, i0_v, i1_v, rows_v, acc_sh, gsem, ssem):
    c = lax.axis_index("c")
    t = lax.axis_index("s")
    rbase = t * _ROWS_PT

    def work(b_hbm, a_hbm, o_hbm):
        @pl.when(t < _DRAIN_T)
        def _():
            pltpu.sync_copy(a_hbm.at[pl.ds(rbase, _ROWS_PT)],
                            acc_sh.at[pl.ds(rbase, _ROWS_PT)])
        plsc.subcore_barrier()

        nch = _SUP // _SV_CHUNK
        total = 2 * nch

        def gather(i, slot):
            j = jnp.where(i < nch, i, i - nch) * _SV_CHUNK
            @pl.when(i < nch)
            def _():
                pltpu.async_copy(
                    b_hbm.at[i1_v.at[pl.ds(j, _SV_CHUNK)]],
                    rows_v.at[slot], gsem)
            @pl.when(i >= nch)
            def _():
                pltpu.async_copy(
                    b_hbm.at[i0_v.at[pl.ds(j, _SV_CHUNK)]],
                    rows_v.at[slot], gsem)

        def scatter(i, slot):
            j = jnp.where(i < nch, i, i - nch) * _SV_CHUNK
            @pl.when(i < nch)
            def _():
                pltpu.async_copy(rows_v.at[slot],
                                 acc_sh.at[i0_v.at[pl.ds(j, _SV_CHUNK)]],
                                 ssem, add=True)
            @pl.when(i >= nch)
            def _():
                pltpu.async_copy(rows_v.at[slot],
                                 acc_sh.at[i1_v.at[pl.ds(j, _SV_CHUNK)]],
                                 ssem, add=True)

        def wait_gather(slot):
            pltpu.make_async_copy(b_hbm.at[i0_v.at[pl.ds(0, _SV_CHUNK)]],
                                  rows_v.at[slot], gsem).wait()

        def wait_scatter(slot):
            pltpu.make_async_copy(rows_v.at[slot],
                                  acc_sh.at[i0_v.at[pl.ds(0, _SV_CHUNK)]],
                                  ssem).wait()

        for s in range(EPT // _SUP):
            base = t * EPT + s * _SUP
            pltpu.sync_copy(e0_hbm.at[pl.ds(base, _SUP)], i0_v)
            pltpu.sync_copy(e1_hbm.at[pl.ds(base, _SUP)], i1_v)
            gather(0, 0)
            gather(1, 1)

            def step(i, _):
                slot = lax.rem(i, 2)
                wait_gather(slot)
                scatter(i, slot)
                @pl.when(i + 2 < total)
                def _():
                    wait_scatter(slot)
                    gather(i + 2, slot)
                return 0

            lax.fori_loop(0, total, step, 0)
            wait_scatter(0)
            wait_scatter(1)

        plsc.subcore_barrier()

        @pl.when(t < _DRAIN_T)
        def _():
            pltpu.sync_copy(acc_sh.at[pl.ds(rbase, _ROWS_PT)],
                            o_hbm.at[pl.ds(rbase, _ROWS_PT)])

    @pl.when(c == 0)
    def _():
        work(bl_hbm, al_hbm, ol_hbm)

    @pl.when(c != 0)
    def _():
        work(br_hbm, ar_hbm, or_hbm)


def _vec_agg(bfeat, e0, e1, init):
    bl, br = bfeat[:, :D // 2], bfeat[:, D // 2:]
    al, ar = init[:, :D // 2], init[:, D // 2:]
    ol, orr = pl.kernel(
        _vec_agg_kernel,
        out_type=[
            jax.ShapeDtypeStruct((N, D // 2), F32),
            jax.ShapeDtypeStruct((N, D // 2), F32),
        ],
        mesh=_mesh(),
        scratch_types=[
            pltpu.VMEM((_SUP,), I32),
            pltpu.VMEM((_SUP,), I32),
            pltpu.VMEM((2, _SV_CHUNK, D // 2), F32),
            pltpu.VMEM_SHARED((N, D // 2), F32),
            pltpu.SemaphoreType.DMA,
            pltpu.SemaphoreType.DMA,
        ],
        compiler_params=pltpu.CompilerParams(use_tc_tiling_on_sc=False),
    )(bl, br, e0, e1, al, ar)
    return ol, orr


def kernel(verts, edges, verts_idx, edges_idx,
           sag0_w0, sag0_b0, sag0_w1, sag0_b1,
           gc0_w0, gc0_b0, gc0_w1, gc0_b1,
           sag1_w0, sag1_b0, sag1_w1, sag1_b1,
           gc1_w0, gc1_b0, gc1_w1, gc1_b1,
           fc1_w, fc1_b, fc2_w, fc2_b):
    x = verts
    e0 = edges[:, 0]
    e1 = edges[:, 1]

    sw01 = jnp.concatenate([sag0_w0, sag0_w1], axis=1)
    sb01 = jnp.concatenate([sag0_b0, sag0_b1])[None, :]
    s0 = _score_mm(x, sw01, sb01)
    parts0 = _scalar_agg(jnp.reshape(s0[:, 1], (N,)), e0, e1)
    a0, b0t, m0 = _select_stage(
        KEEP0, x, s0[:, 0:1], parts0.T,
        gc0_w0, gc0_b0[None, :], gc0_w1, gc0_b1[None, :])

    v0a, v0b = _vec_agg(b0t, e0, e1, a0)
    sw11 = jnp.concatenate([sag1_w0, sag1_w1], axis=1)
    sb11 = jnp.concatenate([sag1_b0, sag1_b1])[None, :]
    h1, s1a, s1bt = _combine_stage(v0a, v0b, sw11, sb11, m0)
    parts1 = _scalar_agg(jnp.reshape(s1bt, (N,)), e0, e1)
    a1, b1t, m1 = _select_stage(
        KEEP1, h1, s1a, parts1.T,
        gc1_w0, gc1_b0[None, :], gc1_w1, gc1_b1[None, :], premask=m0)

    v1a, v1b = _vec_agg(b1t, e0, e1, a1)
    return _head_stage(v1a, v1b, m1, fc1_w, fc1_b[None, :], fc2_w, fc2_b[None, :])
